# Initial kernel scaffold; baseline (speedup 1.0000x reference)
import jax
import jax.numpy as jnp
from jax.experimental import pallas as pl


def kernel(h, gate, edge_index, W_fc, W_attn):
    raise NotImplementedError("write your pallas kernel here")


# speedup vs baseline: 19.4651x; 19.4651x over previous
import functools

import jax
import jax.numpy as jnp
from jax import lax
from jax.experimental import pallas as pl
from jax.experimental.pallas import tpu as pltpu
from jax.experimental.pallas import tpu_sc as plsc

N = 10000
E = 320000
D = 128
DE = 144
NC, NS = 2, 16
NW = NC * NS
EPW = E // NW
K = 80
C = EPW // K
NP = 10112
RPT = NP // NS
G = 25
NSL = C // G
RB = 1000


def _prep_body(h_ref, wt_ref, a1_ref, a2_ref, zext_ref, s_ref):
    hb = h_ref[...]
    zb = jnp.dot(hb, wt_ref[...], preferred_element_type=jnp.float32)
    zext_ref[:, :D] = zb
    zext_ref[:, D:D + 1] = jnp.ones_like(zext_ref[:, D:D + 1])
    zext_ref[:, D + 1:] = jnp.zeros_like(zext_ref[:, D + 1:])
    s_ref[:, 0:1] = jnp.dot(zb, a1_ref[...], preferred_element_type=jnp.float32)
    s_ref[:, 1:2] = jnp.dot(zb, a2_ref[...], preferred_element_type=jnp.float32)
    s_ref[:, 2:] = jnp.zeros_like(s_ref[:, 2:])


_prep = pl.pallas_call(
    _prep_body,
    grid=(N // RB,),
    in_specs=[
        pl.BlockSpec((RB, D), lambda i: (i, 0)),
        pl.BlockSpec((D, D), lambda i: (0, 0)),
        pl.BlockSpec((D, 1), lambda i: (0, 0)),
        pl.BlockSpec((D, 1), lambda i: (0, 0)),
    ],
    out_specs=[
        pl.BlockSpec((RB, DE), lambda i: (i, 0)),
        pl.BlockSpec((RB, 8), lambda i: (i, 0)),
    ],
    out_shape=[
        jax.ShapeDtypeStruct((N, DE), jnp.float32),
        jax.ShapeDtypeStruct((N, 8), jnp.float32),
    ],
)


def _finish_body(u0_ref, u1_ref, o_ref):
    s = u0_ref[...] + u1_ref[...]
    den = s[:, D:D + 1]
    safe = jnp.where(den > 0.0, den, 1.0)
    o_ref[...] = jnp.where(den > 0.0, s[:, :D] / safe, 0.0)


_finish = pl.pallas_call(
    _finish_body,
    grid=(N // RB,),
    in_specs=[
        pl.BlockSpec((RB, DE), lambda i: (i, 0)),
        pl.BlockSpec((RB, DE), lambda i: (i, 0)),
    ],
    out_specs=pl.BlockSpec((RB, D), lambda i: (i, 0)),
    out_shape=jax.ShapeDtypeStruct((N, D), jnp.float32),
)


@functools.partial(
    pl.kernel,
    out_type=jax.ShapeDtypeStruct((NC, NP, DE), jnp.float32),
    mesh=plsc.VectorSubcoreMesh(core_axis_name="c", subcore_axis_name="s"),
    compiler_params=pltpu.CompilerParams(
        needs_layout_passes=False, use_tc_tiling_on_sc=False),
    scratch_types=[
        pltpu.VMEM((G, K), jnp.int32),
        pltpu.VMEM((G, K), jnp.int32),
        pltpu.VMEM((G, K), jnp.float32),
        pltpu.VMEM((N,), jnp.float32),
        pltpu.VMEM((N,), jnp.float32),
        pltpu.VMEM((K, DE), jnp.float32),
        pltpu.VMEM_SHARED((NP, DE), jnp.float32),
        pltpu.SemaphoreType.DMA,
    ],
)
def _sc_edges(src_hbm, dst_hbm, gate_hbm, s1_hbm, s2_hbm, zext_hbm, zrow_hbm,
              out_hbm, src_v, dst_v, gate_v, s1_v, s2_v, rows_v, u_sh, sem):
    cid = lax.axis_index("c")
    sid = lax.axis_index("s")
    wid = sid * NC + cid
    pltpu.sync_copy(s1_hbm, s1_v)
    pltpu.sync_copy(s2_hbm, s2_v)
    pltpu.sync_copy(zrow_hbm, u_sh.at[pl.ds(sid * RPT, RPT)])
    plsc.subcore_barrier()

    def chunk(j, carry):
        gcp = pltpu.async_copy(zext_hbm.at[src_v.at[j]], rows_v, sem)
        exvs = []
        for i in range(K // 16):
            sl = pl.ds(i * 16, 16)
            t = (plsc.load_gather(s1_v, [src_v[j, sl]])
                 + plsc.load_gather(s2_v, [dst_v[j, sl]]))
            t = jnp.where(t >= 0.0, t, 0.01 * t) * gate_v[j, sl]
            exvs.append(jnp.exp(t))
        gcp.wait()
        for g in range(K // 16):
            for eo in range(16):
                e = g * 16 + eo
                w = exvs[g][eo]
                for fb in range(DE // 16):
                    fs = pl.ds(fb * 16, 16)
                    rows_v[e, fs] = rows_v[e, fs] * w
        pltpu.sync_copy(rows_v, u_sh.at[dst_v.at[j]], add=True)
        return carry

    def slab(si, carry):
        pltpu.sync_copy(src_hbm.at[wid, pl.ds(si * G, G)], src_v)
        pltpu.sync_copy(dst_hbm.at[wid, pl.ds(si * G, G)], dst_v)
        pltpu.sync_copy(gate_hbm.at[wid, pl.ds(si * G, G)], gate_v)
        lax.fori_loop(0, G, chunk, 0)
        return carry

    lax.fori_loop(0, NSL, slab, 0)
    plsc.subcore_barrier()
    pltpu.sync_copy(u_sh.at[pl.ds(sid * RPT, RPT)],
                    out_hbm.at[cid, pl.ds(sid * RPT, RPT)])


def kernel(h, gate, edge_index, W_fc, W_attn):
    src = edge_index[0].reshape(NW, C, K)
    dst = edge_index[1].reshape(NW, C, K)
    gate2 = gate.reshape(NW, C, K)
    wt = W_fc.T
    a1 = W_attn[0, :D].reshape(D, 1)
    a2 = W_attn[0, D:].reshape(D, 1)
    zext, s = _prep(h, wt, a1, a2)
    zrow = jnp.zeros((RPT, DE), jnp.float32)
    u = _sc_edges(src, dst, gate2, s[:, 0], s[:, 1], zext, zrow)
    return _finish(u[0], u[1])


# speedup vs baseline: 28.7731x; 1.4782x over previous
import functools

import jax
import jax.numpy as jnp
from jax import lax
from jax.experimental import pallas as pl
from jax.experimental.pallas import tpu as pltpu
from jax.experimental.pallas import tpu_sc as plsc

N = 10000
E = 320000
D = 128
DE = 144
NC, NS = 2, 16
NW = NC * NS
EPW = E // NW
K = 80
C = EPW // K
RPT = N // NS
G = 25
NSL = C // G
RB = 1000


def _prep_body(h_ref, wt_ref, a1_ref, a2_ref, zext_ref, sp_ref):
    wt = wt_ref[...]
    b1 = jnp.dot(wt, a1_ref[...], preferred_element_type=jnp.float32)
    b2 = jnp.dot(wt, a2_ref[...], preferred_element_type=jnp.float32)
    bw = jnp.concatenate(
        [wt, jnp.zeros((D, 1), jnp.float32), b1, b2,
         jnp.zeros((D, DE - D - 3), jnp.float32)], axis=1)
    y = jnp.dot(h_ref[...], bw, preferred_element_type=jnp.float32)
    zext_ref[...] = y
    zext_ref[:, D:D + 1] = jnp.ones_like(zext_ref[:, D:D + 1])
    lo = lax.bitcast_convert_type(
        y[:, D + 1:D + 2].astype(jnp.bfloat16), jnp.uint16).astype(jnp.uint32)
    hi = lax.bitcast_convert_type(
        y[:, D + 2:D + 3].astype(jnp.bfloat16), jnp.uint16).astype(jnp.uint32)
    sp_ref[...] = lax.bitcast_convert_type((hi << 16) | lo, jnp.int32)


_prep = pl.pallas_call(
    _prep_body,
    grid=(N // RB,),
    in_specs=[
        pl.BlockSpec((RB, D), lambda i: (i, 0)),
        pl.BlockSpec((D, D), lambda i: (0, 0)),
        pl.BlockSpec((D, 1), lambda i: (0, 0)),
        pl.BlockSpec((D, 1), lambda i: (0, 0)),
    ],
    out_specs=[
        pl.BlockSpec((RB, DE), lambda i: (i, 0)),
        pl.BlockSpec((RB, 1), lambda i: (i, 0)),
    ],
    out_shape=[
        jax.ShapeDtypeStruct((N, DE), jnp.float32),
        jax.ShapeDtypeStruct((N, 1), jnp.int32),
    ],
)


def _finish_body(u0_ref, u1_ref, o_ref):
    s = u0_ref[0] + u1_ref[0]
    den = s[:, D:D + 1]
    safe = jnp.where(den > 0.0, den, 1.0)
    o_ref[...] = jnp.where(den > 0.0, s[:, :D] / safe, 0.0)


_finish = pl.pallas_call(
    _finish_body,
    grid=(N // RB,),
    in_specs=[
        pl.BlockSpec((1, RB, DE), lambda i: (0, i, 0)),
        pl.BlockSpec((1, RB, DE), lambda i: (1, i, 0)),
    ],
    out_specs=pl.BlockSpec((RB, D), lambda i: (i, 0)),
    out_shape=jax.ShapeDtypeStruct((N, D), jnp.float32),
)


@functools.partial(
    pl.kernel,
    out_type=jax.ShapeDtypeStruct((NC, N, DE), jnp.float32),
    mesh=plsc.VectorSubcoreMesh(core_axis_name="c", subcore_axis_name="s"),
    compiler_params=pltpu.CompilerParams(
        needs_layout_passes=False, use_tc_tiling_on_sc=False),
    scratch_types=[
        pltpu.VMEM((G, K), jnp.int32),
        pltpu.VMEM((G, K), jnp.int32),
        pltpu.VMEM((G, K), jnp.float32),
        pltpu.VMEM((N,), jnp.int32),
        pltpu.VMEM((2, K, DE), jnp.float32),
        pltpu.VMEM_SHARED((N, DE), jnp.float32),
        pltpu.SemaphoreType.DMA((2,)),
        pltpu.SemaphoreType.DMA((2,)),
    ],
)
def _sc_edges(src_hbm, dst_hbm, gate_hbm, sp_hbm, zext_hbm, zrow_hbm,
              out_hbm, src_v, dst_v, gate_v, sp_v, rows_v, u_sh, gsem, ssem):
    cid = lax.axis_index("c")
    sid = lax.axis_index("s")
    wid = sid * NC + cid
    pltpu.sync_copy(sp_hbm, sp_v)
    pltpu.sync_copy(zrow_hbm, u_sh.at[pl.ds(sid * RPT, RPT)])
    plsc.subcore_barrier()

    def chunk(j, si):
        p = lax.rem(j, 2)
        @pl.when(jnp.logical_and(j + 1 < G, j >= 1))
        def _():
            pltpu.make_async_copy(rows_v.at[1 - p], u_sh.at[dst_v.at[j]],
                                  ssem.at[1 - p]).wait()

        @pl.when(j + 1 < G)
        def _():
            pltpu.async_copy(zext_hbm.at[src_v.at[j + 1]],
                             rows_v.at[1 - p], gsem.at[1 - p])

        exvs = []
        for i in range(K // 16):
            sl = pl.ds(i * 16, 16)
            gsrc = plsc.load_gather(sp_v, [src_v[j, sl]])
            gdst = plsc.load_gather(sp_v, [dst_v[j, sl]])
            ts = plsc.bitcast(lax.shift_left(gsrc, 16), jnp.float32)
            td = plsc.bitcast(jnp.bitwise_and(gdst, jnp.int32(-65536)),
                              jnp.float32)
            t = ts + td
            t = jnp.where(t >= 0.0, t, 0.01 * t) * gate_v[j, sl]
            exvs.append(jnp.exp(t))
        pltpu.make_async_copy(zext_hbm.at[src_v.at[j]],
                              rows_v.at[p], gsem.at[p]).wait()
        for g in range(K // 16):
            for eo in range(16):
                e = g * 16 + eo
                w = exvs[g][eo]
                for fb in range(DE // 16):
                    fs = pl.ds(fb * 16, 16)
                    rows_v[p, e, fs] = rows_v[p, e, fs] * w
        pltpu.async_copy(rows_v.at[p], u_sh.at[dst_v.at[j]], ssem.at[p],
                         add=True)
        return si

    def slab(si, carry):
        @pl.when(si >= 1)
        def _():
            pltpu.make_async_copy(rows_v.at[0], u_sh.at[dst_v.at[0]],
                                  ssem.at[0]).wait()
            pltpu.make_async_copy(rows_v.at[1], u_sh.at[dst_v.at[0]],
                                  ssem.at[1]).wait()
        pltpu.sync_copy(src_hbm.at[wid, pl.ds(si * G, G)], src_v)
        pltpu.sync_copy(dst_hbm.at[wid, pl.ds(si * G, G)], dst_v)
        pltpu.sync_copy(gate_hbm.at[wid, pl.ds(si * G, G)], gate_v)
        pltpu.async_copy(zext_hbm.at[src_v.at[0]], rows_v.at[0], gsem.at[0])
        lax.fori_loop(0, G, chunk, si)
        return carry

    lax.fori_loop(0, NSL, slab, 0)
    pltpu.make_async_copy(rows_v.at[0], u_sh.at[dst_v.at[0]],
                          ssem.at[0]).wait()
    pltpu.make_async_copy(rows_v.at[1], u_sh.at[dst_v.at[0]],
                          ssem.at[1]).wait()
    plsc.subcore_barrier()
    pltpu.sync_copy(u_sh.at[pl.ds(sid * RPT, RPT)],
                    out_hbm.at[cid, pl.ds(sid * RPT, RPT)])


def kernel(h, gate, edge_index, W_fc, W_attn):
    src = edge_index[0].reshape(NW, C, K)
    dst = edge_index[1].reshape(NW, C, K)
    gate2 = gate.reshape(NW, C, K)
    wt = W_fc.T
    a1 = W_attn[0, :D].reshape(D, 1)
    a2 = W_attn[0, D:].reshape(D, 1)
    zext, sp2 = _prep(h, wt, a1, a2)
    sp = sp2.reshape(N)
    zrow = jnp.zeros((RPT, DE), jnp.float32)
    u = _sc_edges(src, dst, gate2, sp, zext, zrow)
    return _finish(u, u)


# speedup vs baseline: 29.2162x; 1.0154x over previous
import functools

import jax
import jax.numpy as jnp
from jax import lax
from jax.experimental import pallas as pl
from jax.experimental.pallas import tpu as pltpu
from jax.experimental.pallas import tpu_sc as plsc

N = 10000
E = 320000
D = 128
NC, NS = 2, 16
NW = NC * NS
EPW = E // NW
K = 80
C = EPW // K
RPT = N // NS
G = 25
NSL = C // G
RB = 1000


def _prep_body(h_ref, wt_ref, a1_ref, a2_ref, z_ref, sp_ref):
    wt = wt_ref[...]
    b1 = jnp.dot(wt, a1_ref[...], preferred_element_type=jnp.float32)
    b2 = jnp.dot(wt, a2_ref[...], preferred_element_type=jnp.float32)
    bw = jnp.concatenate(
        [wt, b1, b2, jnp.zeros((D, 14), jnp.float32)], axis=1)
    y = jnp.dot(h_ref[...], bw, preferred_element_type=jnp.float32)
    z_ref[...] = y[:, :D]
    lo = lax.bitcast_convert_type(
        y[:, D:D + 1].astype(jnp.bfloat16), jnp.uint16).astype(jnp.uint32)
    hi = lax.bitcast_convert_type(
        y[:, D + 1:D + 2].astype(jnp.bfloat16), jnp.uint16).astype(jnp.uint32)
    sp_ref[...] = lax.bitcast_convert_type((hi << 16) | lo, jnp.int32)


_prep = pl.pallas_call(
    _prep_body,
    grid=(N // RB,),
    in_specs=[
        pl.BlockSpec((RB, D), lambda i: (i, 0)),
        pl.BlockSpec((D, D), lambda i: (0, 0)),
        pl.BlockSpec((D, 1), lambda i: (0, 0)),
        pl.BlockSpec((D, 1), lambda i: (0, 0)),
    ],
    out_specs=[
        pl.BlockSpec((RB, D), lambda i: (i, 0)),
        pl.BlockSpec((RB, 1), lambda i: (i, 0)),
    ],
    out_shape=[
        jax.ShapeDtypeStruct((N, D), jnp.float32),
        jax.ShapeDtypeStruct((N, 1), jnp.int32),
    ],
)


def _finish_body(u0_ref, u1_ref, d0_ref, d1_ref, o_ref):
    s = u0_ref[0] + u1_ref[0]
    den = d0_ref[0][:, 0:1] + d1_ref[0][:, 0:1]
    safe = jnp.where(den > 0.0, den, 1.0)
    o_ref[...] = jnp.where(den > 0.0, s / safe, 0.0)


_finish = pl.pallas_call(
    _finish_body,
    grid=(N // RB,),
    in_specs=[
        pl.BlockSpec((1, RB, D), lambda i: (0, i, 0)),
        pl.BlockSpec((1, RB, D), lambda i: (1, i, 0)),
        pl.BlockSpec((1, RB, 16), lambda i: (0, i, 0)),
        pl.BlockSpec((1, RB, 16), lambda i: (1, i, 0)),
    ],
    out_specs=pl.BlockSpec((RB, D), lambda i: (i, 0)),
    out_shape=jax.ShapeDtypeStruct((N, D), jnp.float32),
)


@functools.partial(
    pl.kernel,
    out_type=(jax.ShapeDtypeStruct((NC, N, D), jnp.float32),
              jax.ShapeDtypeStruct((NC, N, 16), jnp.float32)),
    mesh=plsc.VectorSubcoreMesh(core_axis_name="c", subcore_axis_name="s"),
    compiler_params=pltpu.CompilerParams(
        needs_layout_passes=False, use_tc_tiling_on_sc=False),
    scratch_types=[
        pltpu.VMEM((G, K), jnp.int32),
        pltpu.VMEM((G, K), jnp.int32),
        pltpu.VMEM((G, K), jnp.float32),
        pltpu.VMEM((N,), jnp.int32),
        pltpu.VMEM((2, K, D), jnp.float32),
        pltpu.VMEM((2, K, 16), jnp.float32),
        pltpu.VMEM_SHARED((N, D), jnp.float32),
        pltpu.VMEM_SHARED((N, 16), jnp.float32),
        pltpu.SemaphoreType.DMA((2,)),
        pltpu.SemaphoreType.DMA((2,)),
        pltpu.SemaphoreType.DMA((2,)),
    ],
)
def _sc_edges(ei_hbm, gate_hbm, sp_hbm, z_hbm, zrow_hbm, zden_hbm,
              u_hbm, den_hbm, src_v, dst_v, gate_v, sp_v, rows_v, ex_v,
              u_sh, den_sh, gsem, ssem, dsem):
    cid = lax.axis_index("c")
    sid = lax.axis_index("s")
    wid = sid * NC + cid
    pltpu.sync_copy(sp_hbm, sp_v)
    pltpu.sync_copy(zrow_hbm, u_sh.at[pl.ds(sid * RPT, RPT)])
    pltpu.sync_copy(zden_hbm, den_sh.at[pl.ds(sid * RPT, RPT)])
    plsc.subcore_barrier()

    def chunk(j, si):
        p = lax.rem(j, 2)
        @pl.when(jnp.logical_and(j + 1 < G, j >= 1))
        def _():
            pltpu.make_async_copy(rows_v.at[1 - p], u_sh.at[dst_v.at[j]],
                                  ssem.at[1 - p]).wait()

        @pl.when(j + 1 < G)
        def _():
            pltpu.async_copy(z_hbm.at[src_v.at[j + 1]],
                             rows_v.at[1 - p], gsem.at[1 - p])

        @pl.when(j >= 2)
        def _():
            pltpu.make_async_copy(ex_v.at[p], den_sh.at[dst_v.at[j]],
                                  dsem.at[p]).wait()

        exvs = []
        for i in range(K // 16):
            sl = pl.ds(i * 16, 16)
            gsrc = plsc.load_gather(sp_v, [src_v[j, sl]])
            gdst = plsc.load_gather(sp_v, [dst_v[j, sl]])
            ts = plsc.bitcast(lax.shift_left(gsrc, 16), jnp.float32)
            td = plsc.bitcast(jnp.bitwise_and(gdst, jnp.int32(-65536)),
                              jnp.float32)
            t = ts + td
            t = jnp.where(t >= 0.0, t, 0.01 * t) * gate_v[j, sl]
            exvs.append(jnp.exp(t))
        pltpu.make_async_copy(z_hbm.at[src_v.at[j]],
                              rows_v.at[p], gsem.at[p]).wait()
        for g in range(K // 16):
            for eo in range(16):
                e = g * 16 + eo
                w = exvs[g][eo]
                ex_v[p, e, :] = jnp.full((16,), w, jnp.float32)
                for fb in range(D // 16):
                    fs = pl.ds(fb * 16, 16)
                    rows_v[p, e, fs] = rows_v[p, e, fs] * w
        pltpu.async_copy(rows_v.at[p], u_sh.at[dst_v.at[j]], ssem.at[p],
                         add=True)
        pltpu.async_copy(ex_v.at[p], den_sh.at[dst_v.at[j]], dsem.at[p],
                         add=True)
        return si

    def slab(si, carry):
        @pl.when(si >= 1)
        def _():
            pltpu.make_async_copy(rows_v.at[0], u_sh.at[dst_v.at[0]],
                                  ssem.at[0]).wait()
            pltpu.make_async_copy(rows_v.at[1], u_sh.at[dst_v.at[0]],
                                  ssem.at[1]).wait()
            pltpu.make_async_copy(ex_v.at[0], den_sh.at[dst_v.at[0]],
                                  dsem.at[0]).wait()
            pltpu.make_async_copy(ex_v.at[1], den_sh.at[dst_v.at[0]],
                                  dsem.at[1]).wait()
        pltpu.sync_copy(ei_hbm.at[0, wid, pl.ds(si * G, G)], src_v)
        pltpu.sync_copy(ei_hbm.at[1, wid, pl.ds(si * G, G)], dst_v)
        pltpu.sync_copy(gate_hbm.at[wid, pl.ds(si * G, G)], gate_v)
        pltpu.async_copy(z_hbm.at[src_v.at[0]], rows_v.at[0], gsem.at[0])
        lax.fori_loop(0, G, chunk, si)
        return carry

    lax.fori_loop(0, NSL, slab, 0)
    pltpu.make_async_copy(rows_v.at[0], u_sh.at[dst_v.at[0]],
                          ssem.at[0]).wait()
    pltpu.make_async_copy(rows_v.at[1], u_sh.at[dst_v.at[0]],
                          ssem.at[1]).wait()
    pltpu.make_async_copy(ex_v.at[0], den_sh.at[dst_v.at[0]],
                          dsem.at[0]).wait()
    pltpu.make_async_copy(ex_v.at[1], den_sh.at[dst_v.at[0]],
                          dsem.at[1]).wait()
    plsc.subcore_barrier()
    pltpu.sync_copy(u_sh.at[pl.ds(sid * RPT, RPT)],
                    u_hbm.at[cid, pl.ds(sid * RPT, RPT)])
    pltpu.sync_copy(den_sh.at[pl.ds(sid * RPT, RPT)],
                    den_hbm.at[cid, pl.ds(sid * RPT, RPT)])


def kernel(h, gate, edge_index, W_fc, W_attn):
    ei = edge_index.reshape(2, NW, C, K)
    gate2 = gate.reshape(NW, C, K)
    wt = W_fc.T
    a1 = W_attn[0, :D].reshape(D, 1)
    a2 = W_attn[0, D:].reshape(D, 1)
    z, sp2 = _prep(h, wt, a1, a2)
    sp = sp2.reshape(N)
    zrow = jnp.zeros((RPT, D), jnp.float32)
    zden = jnp.zeros((RPT, 16), jnp.float32)
    u, den = _sc_edges(ei, gate2, sp, z, zrow, zden)
    return _finish(u, u, den, den)


# speedup vs baseline: 37.4810x; 1.2829x over previous
import functools

import jax
import jax.numpy as jnp
from jax import lax
from jax.experimental import pallas as pl
from jax.experimental.pallas import tpu as pltpu
from jax.experimental.pallas import tpu_sc as plsc

N = 10000
E = 320000
D = 128
NC, NS = 2, 16
NW = NC * NS
EPW = E // NW
K = 80
C = EPW // K
RPT = N // NS
G = 25
NSL = C // G
RB = 1000
ND = 10112
RPD = ND // NS
NB = 3
TAIL = sorted({(G - 3) % NB, (G - 2) % NB, (G - 1) % NB})


def _prep_body(h_ref, wt_ref, a1_ref, a2_ref, z_ref, sp_ref):
    wt = wt_ref[...]
    b1 = jnp.dot(wt, a1_ref[...], preferred_element_type=jnp.float32)
    b2 = jnp.dot(wt, a2_ref[...], preferred_element_type=jnp.float32)
    bw = jnp.concatenate(
        [wt, b1, b2, jnp.zeros((D, 14), jnp.float32)], axis=1)
    y = jnp.dot(h_ref[...], bw, preferred_element_type=jnp.float32)
    z_ref[...] = y[:, :D]
    lo = lax.bitcast_convert_type(
        y[:, D:D + 1].astype(jnp.bfloat16), jnp.uint16).astype(jnp.uint32)
    hi = lax.bitcast_convert_type(
        y[:, D + 1:D + 2].astype(jnp.bfloat16), jnp.uint16).astype(jnp.uint32)
    sp_ref[...] = lax.bitcast_convert_type((hi << 16) | lo, jnp.int32)


_prep = pl.pallas_call(
    _prep_body,
    grid=(N // RB,),
    in_specs=[
        pl.BlockSpec((RB, D), lambda i: (i, 0)),
        pl.BlockSpec((D, D), lambda i: (0, 0)),
        pl.BlockSpec((D, 1), lambda i: (0, 0)),
        pl.BlockSpec((D, 1), lambda i: (0, 0)),
    ],
    out_specs=[
        pl.BlockSpec((RB, D), lambda i: (i, 0)),
        pl.BlockSpec((RB, 1), lambda i: (i, 0)),
    ],
    out_shape=[
        jax.ShapeDtypeStruct((N, D), jnp.float32),
        jax.ShapeDtypeStruct((N, 1), jnp.int32),
    ],
)


def _finish_body(u0_ref, u1_ref, den_ref, o_ref):
    s = u0_ref[0] + u1_ref[0]
    den = den_ref[...]
    safe = jnp.where(den > 0.0, den, 1.0)
    o_ref[...] = jnp.where(den > 0.0, s / safe, 0.0)


_finish = pl.pallas_call(
    _finish_body,
    grid=(N // RB,),
    in_specs=[
        pl.BlockSpec((1, RB, D), lambda i: (0, i, 0)),
        pl.BlockSpec((1, RB, D), lambda i: (1, i, 0)),
        pl.BlockSpec((RB, 1), lambda i: (i, 0)),
    ],
    out_specs=pl.BlockSpec((RB, D), lambda i: (i, 0)),
    out_shape=jax.ShapeDtypeStruct((N, D), jnp.float32),
)


@functools.partial(
    pl.kernel,
    out_type=(jax.ShapeDtypeStruct((NC, N, D), jnp.float32),
              jax.ShapeDtypeStruct((NC, ND), jnp.float32)),
    mesh=plsc.VectorSubcoreMesh(core_axis_name="c", subcore_axis_name="s"),
    compiler_params=pltpu.CompilerParams(
        needs_layout_passes=False, use_tc_tiling_on_sc=False),
    scratch_types=[
        pltpu.VMEM((G, K), jnp.int32),
        pltpu.VMEM((G, K), jnp.int32),
        pltpu.VMEM((G, K), jnp.float32),
        pltpu.VMEM((N,), jnp.int32),
        pltpu.VMEM((NB, K, D), jnp.float32),
        pltpu.VMEM((NB, K), jnp.float32),
        pltpu.VMEM_SHARED((N, D), jnp.float32),
        pltpu.VMEM_SHARED((ND,), jnp.float32),
        pltpu.SemaphoreType.DMA((NB,)),
        pltpu.SemaphoreType.DMA((NB,)),
        pltpu.SemaphoreType.DMA((NB,)),
    ],
)
def _sc_edges(ei_hbm, gate_hbm, sp_hbm, z_hbm, zrow_hbm, zden_hbm,
              u_hbm, den_hbm, src_v, dst_v, gate_v, sp_v, rows_v, ex_v,
              u_sh, den_sh, gsem, ssem, dsem):
    cid = lax.axis_index("c")
    sid = lax.axis_index("s")
    wid = sid * NC + cid
    pltpu.sync_copy(sp_hbm, sp_v)
    pltpu.sync_copy(zrow_hbm, u_sh.at[pl.ds(sid * RPT, RPT)])
    pltpu.sync_copy(zden_hbm, den_sh.at[pl.ds(sid * RPD, RPD)])
    plsc.subcore_barrier()

    def chunk(j, si):
        b = lax.rem(j, NB)
        nb = lax.rem(j + 1, NB)
        @pl.when(jnp.logical_and(j + 1 < G, j >= 2))
        def _():
            pltpu.make_async_copy(rows_v.at[nb], u_sh.at[dst_v.at[j]],
                                  ssem.at[nb]).wait()
            pltpu.make_async_copy(ex_v.at[nb], den_sh.at[dst_v.at[j]],
                                  dsem.at[nb]).wait()

        @pl.when(j + 1 < G)
        def _():
            pltpu.async_copy(z_hbm.at[src_v.at[j + 1]],
                             rows_v.at[nb], gsem.at[nb])

        exvs = []
        for i in range(K // 16):
            sl = pl.ds(i * 16, 16)
            gsrc = plsc.load_gather(sp_v, [src_v[j, sl]])
            gdst = plsc.load_gather(sp_v, [dst_v[j, sl]])
            ts = plsc.bitcast(lax.shift_left(gsrc, 16), jnp.float32)
            td = plsc.bitcast(jnp.bitwise_and(gdst, jnp.int32(-65536)),
                              jnp.float32)
            t = ts + td
            t = jnp.where(t >= 0.0, t, 0.01 * t) * gate_v[j, sl]
            ex = jnp.exp(t)
            ex_v[b, sl] = ex
            exvs.append(ex)
        pltpu.make_async_copy(z_hbm.at[src_v.at[j]],
                              rows_v.at[b], gsem.at[b]).wait()
        for g in range(K // 16):
            for eo in range(16):
                e = g * 16 + eo
                w = exvs[g][eo]
                for fb in range(D // 16):
                    fs = pl.ds(fb * 16, 16)
                    rows_v[b, e, fs] = rows_v[b, e, fs] * w
        pltpu.async_copy(rows_v.at[b], u_sh.at[dst_v.at[j]], ssem.at[b],
                         add=True)
        pltpu.async_copy(ex_v.at[b], den_sh.at[dst_v.at[j]], dsem.at[b],
                         add=True)
        return si

    def slab(si, carry):
        @pl.when(si >= 1)
        def _():
            for b in TAIL:
                pltpu.make_async_copy(rows_v.at[b], u_sh.at[dst_v.at[0]],
                                      ssem.at[b]).wait()
                pltpu.make_async_copy(ex_v.at[b], den_sh.at[dst_v.at[0]],
                                      dsem.at[b]).wait()
        pltpu.sync_copy(ei_hbm.at[0, wid, pl.ds(si * G, G)], src_v)
        pltpu.sync_copy(ei_hbm.at[1, wid, pl.ds(si * G, G)], dst_v)
        pltpu.sync_copy(gate_hbm.at[wid, pl.ds(si * G, G)], gate_v)
        pltpu.async_copy(z_hbm.at[src_v.at[0]], rows_v.at[0], gsem.at[0])
        lax.fori_loop(0, G, chunk, si)
        return carry

    lax.fori_loop(0, NSL, slab, 0)
    for b in TAIL:
        pltpu.make_async_copy(rows_v.at[b], u_sh.at[dst_v.at[0]],
                              ssem.at[b]).wait()
        pltpu.make_async_copy(ex_v.at[b], den_sh.at[dst_v.at[0]],
                              dsem.at[b]).wait()
    plsc.subcore_barrier()
    pltpu.sync_copy(u_sh.at[pl.ds(sid * RPT, RPT)],
                    u_hbm.at[cid, pl.ds(sid * RPT, RPT)])
    pltpu.sync_copy(den_sh.at[pl.ds(sid * RPD, RPD)],
                    den_hbm.at[cid, pl.ds(sid * RPD, RPD)])


def kernel(h, gate, edge_index, W_fc, W_attn):
    ei = edge_index.reshape(2, NW, C, K)
    gate2 = gate.reshape(NW, C, K)
    wt = W_fc.T
    a1 = W_attn[0, :D].reshape(D, 1)
    a2 = W_attn[0, D:].reshape(D, 1)
    z, sp2 = _prep(h, wt, a1, a2)
    sp = sp2.reshape(N)
    zrow = jnp.zeros((RPT, D), jnp.float32)
    zden = jnp.zeros((RPD,), jnp.float32)
    u, den = _sc_edges(ei, gate2, sp, z, zrow, zden)
    den_sum = (den[0, :N] + den[1, :N]).reshape(N, 1)
    return _finish(u, u, den_sum)
